# baseline (speedup 1.0000x reference)
import functools

import jax
import jax.numpy as jnp
from jax import lax
from jax.experimental import pallas as pl
from jax.experimental.pallas import tpu as pltpu
from jax.experimental.pallas import tpu_sc as plsc

N = 10000
E = 320000
H = 128
OUT = 2
NUM_LAYERS = 4
EPS = 1e-5

GROUP = 128
NGROUPS = E // GROUP
NWORK = 32
ITERS = -(-NGROUPS // NWORK)
NSUB = 16
NPAD = 10240
ROWS_PER_SUB = NPAD // NSUB
DEGW = 16

BR = 400


HH = H // 2

@functools.lru_cache(maxsize=None)
def _make_agg_sc():
    mesh = plsc.VectorSubcoreMesh(core_axis_name="c", subcore_axis_name="s")
    NB = 4
    GPW = 160

    @functools.partial(
        pl.kernel,
        out_type=jax.ShapeDtypeStruct((2, NPAD, HH), jnp.float32),
        scratch_types=[
            pltpu.VMEM((NB, 2, GROUP), jnp.int32),
            pltpu.VMEM((NB, GROUP, HH), jnp.float32),
            pltpu.VMEM_SHARED((NPAD, HH), jnp.float32),
            pltpu.VMEM_SHARED((NPAD, HH), jnp.float32),
        ] + [pltpu.SemaphoreType.DMA] * NB,
        mesh=mesh,
        compiler_params=pltpu.CompilerParams(use_tc_tiling_on_sc=False),
    )
    def agg(u_hbm, ei_hbm, out_hbm, idx_v, rows_v, u_sh, acc_sh, *sems):
        c = lax.axis_index("c")
        s = lax.axis_index("s")
        g0 = s * GPW
        nw = jnp.clip(NGROUPS - g0, 0, GPW)

        stripe = pl.ds(s * ROWS_PER_SUB, ROWS_PER_SUB)
        pltpu.sync_copy(u_hbm.at[c, stripe], u_sh.at[stripe])
        pltpu.sync_copy(u_hbm.at[c, stripe], acc_sh.at[stripe])
        plsc.subcore_barrier()

        def start(b, i):
            g = g0 + i
            pltpu.sync_copy(ei_hbm.at[:, pl.ds(g * GROUP, GROUP)], idx_v.at[b])
            pltpu.async_copy(u_sh.at[idx_v.at[b].at[0]], rows_v.at[b], sems[b])

        def wait(b):
            pltpu.make_async_copy(
                u_sh.at[pl.ds(0, GROUP)], rows_v.at[b], sems[b]).wait()

        def scat(b):
            pltpu.sync_copy(rows_v.at[b], acc_sh.at[idx_v.at[b].at[1]], add=True)

        for j in range(NB):
            @pl.when(j < nw)
            def _(j=j):
                start(j, j)

        def body(k, _):
            base = k * NB
            for b in range(NB):
                i = base + b

                @pl.when(i < nw)
                def _(b=b, i=i):
                    wait(b)
                    scat(b)

                    @pl.when(i + NB < nw)
                    def _(b=b, i=i):
                        start(b, i + NB)

            return ()

        lax.fori_loop(0, GPW // NB, body, ())
        plsc.subcore_barrier()
        pltpu.sync_copy(acc_sh.at[stripe], out_hbm.at[c, stripe])

    return agg


def _agg_sc(u2, ei):
    return _make_agg_sc()(u2, ei)


@functools.lru_cache(maxsize=None)
def _make_deg_sc():
    mesh = plsc.VectorSubcoreMesh(core_axis_name="c", subcore_axis_name="s")

    @functools.partial(
        pl.kernel,
        out_type=jax.ShapeDtypeStruct((2, NPAD, DEGW), jnp.float32),
        scratch_types=[
            pltpu.VMEM((2, GROUP), jnp.int32),
            pltpu.VMEM((GROUP, DEGW), jnp.float32),
            pltpu.VMEM_SHARED((NPAD, DEGW), jnp.float32),
        ],
        mesh=mesh,
        compiler_params=pltpu.CompilerParams(use_tc_tiling_on_sc=False),
    )
    def degk(ei_hbm, ones_hbm, zeros_hbm, out_hbm, idx_v, ones_v, acc_sh):
        c = lax.axis_index("c")
        s = lax.axis_index("s")
        w = s * 2 + c
        pltpu.sync_copy(zeros_hbm, acc_sh.at[pl.ds(s * ROWS_PER_SUB, ROWS_PER_SUB)])
        pltpu.sync_copy(ones_hbm, ones_v)
        plsc.subcore_barrier()

        def body(i, _):
            g = i * NWORK + w

            @pl.when(g < NGROUPS)
            def _():
                pltpu.sync_copy(ei_hbm.at[:, pl.ds(g * GROUP, GROUP)], idx_v)
                pltpu.sync_copy(ones_v, acc_sh.at[idx_v.at[1]], add=True)

            return ()

        lax.fori_loop(0, ITERS, body, ())
        plsc.subcore_barrier()
        pltpu.sync_copy(
            acc_sh.at[pl.ds(s * ROWS_PER_SUB, ROWS_PER_SUB)],
            out_hbm.at[c, pl.ds(s * ROWS_PER_SUB, ROWS_PER_SUB)],
        )

    return degk


def _deg_sc(ei, ones_d, zeros_d):
    return _make_deg_sc()(ei, ones_d, zeros_d)


def _mm_body(h_ref, w_ref, b_ref, s_ref, o_ref):
    acc = jnp.dot(h_ref[...], w_ref[...], preferred_element_type=jnp.float32)
    o_ref[...] = (acc + b_ref[...]) * s_ref[...]


def _mm(h, w, b, scale):
    return pl.pallas_call(
        _mm_body,
        grid=(N // BR,),
        in_specs=[
            pl.BlockSpec((BR, H), lambda i: (i, 0)),
            pl.BlockSpec((H, H), lambda i: (0, 0)),
            pl.BlockSpec((1, H), lambda i: (0, 0)),
            pl.BlockSpec((BR, 1), lambda i: (i, 0)),
        ],
        out_specs=pl.BlockSpec((BR, H), lambda i: (i, 0)),
        out_shape=jax.ShapeDtypeStruct((N, H), jnp.float32),
    )(h, w, b, scale)


def _mmu_body(h_ref, w_ref, s_ref, o_ref):
    acc = jnp.dot(h_ref[...], w_ref[...], preferred_element_type=jnp.float32)
    acc = acc * s_ref[...]
    o_ref[0] = acc[:, :HH]
    o_ref[1] = acc[:, HH:]


def _mmu(h, w, scale):
    return pl.pallas_call(
        _mmu_body,
        grid=(N // BR,),
        in_specs=[
            pl.BlockSpec((BR, H), lambda i: (i, 0)),
            pl.BlockSpec((H, H), lambda i: (0, 0)),
            pl.BlockSpec((BR, 1), lambda i: (i, 0)),
        ],
        out_specs=pl.BlockSpec((2, BR, HH), lambda i: (0, i, 0)),
        out_shape=jax.ShapeDtypeStruct((2, NPAD, HH), jnp.float32),
    )(h, w, scale)


def _dis_body(d_ref, o_ref):
    deg = 1.0 + d_ref[0, :, 0:1] + d_ref[1, :, 0:1]
    o_ref[...] = lax.rsqrt(deg)


def _dis(degp):
    return pl.pallas_call(
        _dis_body,
        grid=(N // BR,),
        in_specs=[pl.BlockSpec((2, BR, DEGW), lambda i: (0, i, 0))],
        out_specs=pl.BlockSpec((BR, 1), lambda i: (i, 0)),
        out_shape=jax.ShapeDtypeStruct((N, 1), jnp.float32),
    )(degp)


def _post_body(p_ref, dis_ref, b_ref, g_ref, bb_ref, r_ref, o_ref):
    t = jnp.concatenate([p_ref[0], p_ref[1]], axis=-1) * dis_ref[...] + b_ref[...]
    mu = jnp.mean(t, axis=-1, keepdims=True)
    d = t - mu
    var = jnp.mean(d * d, axis=-1, keepdims=True)
    y = d * lax.rsqrt(var + EPS) * g_ref[...] + bb_ref[...]
    o_ref[...] = jnp.maximum(y, 0.0) + r_ref[...]


def _post(p, dis, b, g, bb, r):
    return pl.pallas_call(
        _post_body,
        grid=(N // BR,),
        in_specs=[
            pl.BlockSpec((2, BR, HH), lambda i: (0, i, 0)),
            pl.BlockSpec((BR, 1), lambda i: (i, 0)),
            pl.BlockSpec((1, H), lambda i: (0, 0)),
            pl.BlockSpec((1, H), lambda i: (0, 0)),
            pl.BlockSpec((1, H), lambda i: (0, 0)),
            pl.BlockSpec((BR, H), lambda i: (i, 0)),
        ],
        out_specs=pl.BlockSpec((BR, H), lambda i: (i, 0)),
        out_shape=jax.ShapeDtypeStruct((N, H), jnp.float32),
    )(p, dis, b, g, bb, r)


def _sum_body(h_ref, o_ref):
    @pl.when(pl.program_id(0) == 0)
    def _():
        o_ref[...] = jnp.zeros_like(o_ref)

    o_ref[...] += jnp.sum(h_ref[...], axis=0, keepdims=True)


def _colsum(h):
    return pl.pallas_call(
        _sum_body,
        grid=(N // BR,),
        in_specs=[pl.BlockSpec((BR, H), lambda i: (i, 0))],
        out_specs=pl.BlockSpec((1, H), lambda i: (0, 0)),
        out_shape=jax.ShapeDtypeStruct((1, H), jnp.float32),
    )(h)


def _head_body(m_ref, w1_ref, b1_ref, g_ref, b_ref, w2_ref, b2_ref, o_ref):
    m = m_ref[...] * (1.0 / N)
    t = jnp.dot(m, w1_ref[...], preferred_element_type=jnp.float32) + b1_ref[...]
    mu = jnp.mean(t, axis=-1, keepdims=True)
    d = t - mu
    var = jnp.mean(d * d, axis=-1, keepdims=True)
    y = d * lax.rsqrt(var + EPS) * g_ref[...] + b_ref[...]
    y = jnp.maximum(y, 0.0)
    o_ref[...] = jnp.dot(y, w2_ref[...], preferred_element_type=jnp.float32) + b2_ref[...]


def _head(m, w1, b1, g, b, w2, b2):
    return pl.pallas_call(
        _head_body,
        grid=(1,),
        in_specs=[
            pl.BlockSpec((1, H), lambda i: (0, 0)),
            pl.BlockSpec((H, H), lambda i: (0, 0)),
            pl.BlockSpec((1, H), lambda i: (0, 0)),
            pl.BlockSpec((1, H), lambda i: (0, 0)),
            pl.BlockSpec((1, H), lambda i: (0, 0)),
            pl.BlockSpec((H, OUT), lambda i: (0, 0)),
            pl.BlockSpec((1, OUT), lambda i: (0, 0)),
        ],
        out_specs=pl.BlockSpec((1, OUT), lambda i: (0, 0)),
        out_shape=jax.ShapeDtypeStruct((1, OUT), jnp.float32),
    )(m, w1, b1, g, b, w2, b2)


def kernel(x, edge_index, params):
    p = params
    ei = edge_index.astype(jnp.int32)

    zeros_d = jnp.zeros((ROWS_PER_SUB, DEGW), jnp.float32)
    ones_d = jnp.ones((GROUP, DEGW), jnp.float32)
    ones_n = jnp.ones((N, 1), jnp.float32)

    degp = _deg_sc(ei, ones_d, zeros_d)
    dis = _dis(degp)

    h = _mm(x, p["emb_W"], p["emb_b"][None], ones_n)
    for i in range(NUM_LAYERS):
        if i % 2 == 0 and i > 0:
            r = _mm(h, p["res_W"], p["res_b"][None], ones_n)
        else:
            r = h
        u2 = _mmu(h, p["conv_W"][i], dis)
        agg = _agg_sc(u2, ei)
        h = _post(agg, dis, p["conv_b"][i][None], p["ln_g"][i][None],
                  p["ln_b"][i][None], r)

    hs = _colsum(h)
    out = _head(hs, p["fc1_W"], p["fc1_b"][None], p["fcn_g"][None],
                p["fcn_b"][None], p["fc2_W"], p["fc2_b"][None])
    return out


# speedup vs baseline: 1.5093x; 1.5093x over previous
import functools

import jax
import jax.numpy as jnp
from jax import lax
from jax.experimental import pallas as pl
from jax.experimental.pallas import tpu as pltpu
from jax.experimental.pallas import tpu_sc as plsc

N = 10000
E = 320000
H = 128
OUT = 2
NUM_LAYERS = 4
EPS = 1e-5

GROUP = 128
NGROUPS = E // GROUP
NWORK = 32
ITERS = -(-NGROUPS // NWORK)
NSUB = 16
NPAD = 10240
ROWS_PER_SUB = NPAD // NSUB
DEGW = 16

BR = 400


HH = H // 2

@functools.lru_cache(maxsize=None)
def _make_agg_sc():
    mesh = plsc.VectorSubcoreMesh(core_axis_name="c", subcore_axis_name="s")
    NB = 2
    G2 = 512
    NG2 = E // G2
    GPW = 40

    @functools.partial(
        pl.kernel,
        out_type=jax.ShapeDtypeStruct((2, NPAD, HH), jnp.float32),
        scratch_types=[
            pltpu.VMEM((NB, 2, G2), jnp.int32),
            pltpu.VMEM((NB, G2, HH), jnp.float32),
            pltpu.VMEM_SHARED((NPAD, HH), jnp.float32),
        ] + [pltpu.SemaphoreType.DMA] * NB,
        mesh=mesh,
        compiler_params=pltpu.CompilerParams(use_tc_tiling_on_sc=False),
    )
    def agg(u_hbm, ei_hbm, out_hbm, idx_v, rows_v, acc_sh, *sems):
        c = lax.axis_index("c")
        s = lax.axis_index("s")
        g0 = s * GPW
        nw = jnp.clip(NG2 - g0, 0, GPW)

        stripe = pl.ds(s * ROWS_PER_SUB, ROWS_PER_SUB)
        pltpu.sync_copy(u_hbm.at[c, stripe], acc_sh.at[stripe])
        plsc.subcore_barrier()

        def start(b, i):
            g = g0 + i
            pltpu.sync_copy(ei_hbm.at[:, pl.ds(g * G2, G2)], idx_v.at[b])

            @pl.when(c == 0)
            def _():
                pltpu.async_copy(u_hbm.at[0].at[idx_v.at[b].at[0]],
                                 rows_v.at[b], sems[b])

            @pl.when(c == 1)
            def _():
                pltpu.async_copy(u_hbm.at[1].at[idx_v.at[b].at[0]],
                                 rows_v.at[b], sems[b])

        def wait(b):
            pltpu.make_async_copy(
                u_hbm.at[0].at[pl.ds(0, G2)], rows_v.at[b], sems[b]).wait()

        def scat(b):
            pltpu.sync_copy(rows_v.at[b], acc_sh.at[idx_v.at[b].at[1]], add=True)

        for j in range(NB):
            @pl.when(j < nw)
            def _(j=j):
                start(j, j)

        def body(k, _):
            base = k * NB
            for b in range(NB):
                i = base + b

                @pl.when(i < nw)
                def _(b=b, i=i):
                    wait(b)
                    scat(b)

                    @pl.when(i + NB < nw)
                    def _(b=b, i=i):
                        start(b, i + NB)

            return ()

        lax.fori_loop(0, GPW // NB, body, ())
        plsc.subcore_barrier()
        pltpu.sync_copy(acc_sh.at[stripe], out_hbm.at[c, stripe])

    return agg


def _agg_sc(u2, ei):
    return _make_agg_sc()(u2, ei)


@functools.lru_cache(maxsize=None)
def _make_deg_sc():
    mesh = plsc.VectorSubcoreMesh(core_axis_name="c", subcore_axis_name="s")

    @functools.partial(
        pl.kernel,
        out_type=jax.ShapeDtypeStruct((2, NPAD, DEGW), jnp.float32),
        scratch_types=[
            pltpu.VMEM((2, GROUP), jnp.int32),
            pltpu.VMEM((GROUP, DEGW), jnp.float32),
            pltpu.VMEM_SHARED((NPAD, DEGW), jnp.float32),
        ],
        mesh=mesh,
        compiler_params=pltpu.CompilerParams(use_tc_tiling_on_sc=False),
    )
    def degk(ei_hbm, ones_hbm, zeros_hbm, out_hbm, idx_v, ones_v, acc_sh):
        c = lax.axis_index("c")
        s = lax.axis_index("s")
        w = s * 2 + c
        pltpu.sync_copy(zeros_hbm, acc_sh.at[pl.ds(s * ROWS_PER_SUB, ROWS_PER_SUB)])
        pltpu.sync_copy(ones_hbm, ones_v)
        plsc.subcore_barrier()

        def body(i, _):
            g = i * NWORK + w

            @pl.when(g < NGROUPS)
            def _():
                pltpu.sync_copy(ei_hbm.at[:, pl.ds(g * GROUP, GROUP)], idx_v)
                pltpu.sync_copy(ones_v, acc_sh.at[idx_v.at[1]], add=True)

            return ()

        lax.fori_loop(0, ITERS, body, ())
        plsc.subcore_barrier()
        pltpu.sync_copy(
            acc_sh.at[pl.ds(s * ROWS_PER_SUB, ROWS_PER_SUB)],
            out_hbm.at[c, pl.ds(s * ROWS_PER_SUB, ROWS_PER_SUB)],
        )

    return degk


def _deg_sc(ei, ones_d, zeros_d):
    return _make_deg_sc()(ei, ones_d, zeros_d)


def _mm_body(h_ref, w_ref, b_ref, s_ref, o_ref):
    acc = jnp.dot(h_ref[...], w_ref[...], preferred_element_type=jnp.float32)
    o_ref[...] = (acc + b_ref[...]) * s_ref[...]


def _mm(h, w, b, scale):
    return pl.pallas_call(
        _mm_body,
        grid=(N // BR,),
        in_specs=[
            pl.BlockSpec((BR, H), lambda i: (i, 0)),
            pl.BlockSpec((H, H), lambda i: (0, 0)),
            pl.BlockSpec((1, H), lambda i: (0, 0)),
            pl.BlockSpec((BR, 1), lambda i: (i, 0)),
        ],
        out_specs=pl.BlockSpec((BR, H), lambda i: (i, 0)),
        out_shape=jax.ShapeDtypeStruct((N, H), jnp.float32),
    )(h, w, b, scale)


def _mmu_body(h_ref, w_ref, s_ref, o_ref):
    acc = jnp.dot(h_ref[...], w_ref[...], preferred_element_type=jnp.float32)
    acc = acc * s_ref[...]
    o_ref[0] = acc[:, :HH]
    o_ref[1] = acc[:, HH:]


def _mmu(h, w, scale):
    return pl.pallas_call(
        _mmu_body,
        grid=(N // BR,),
        in_specs=[
            pl.BlockSpec((BR, H), lambda i: (i, 0)),
            pl.BlockSpec((H, H), lambda i: (0, 0)),
            pl.BlockSpec((BR, 1), lambda i: (i, 0)),
        ],
        out_specs=pl.BlockSpec((2, BR, HH), lambda i: (0, i, 0)),
        out_shape=jax.ShapeDtypeStruct((2, NPAD, HH), jnp.float32),
    )(h, w, scale)


def _dis_body(d_ref, o_ref):
    deg = 1.0 + d_ref[0, :, 0:1] + d_ref[1, :, 0:1]
    o_ref[...] = lax.rsqrt(deg)


def _dis(degp):
    return pl.pallas_call(
        _dis_body,
        grid=(N // BR,),
        in_specs=[pl.BlockSpec((2, BR, DEGW), lambda i: (0, i, 0))],
        out_specs=pl.BlockSpec((BR, 1), lambda i: (i, 0)),
        out_shape=jax.ShapeDtypeStruct((N, 1), jnp.float32),
    )(degp)


def _post_body(p_ref, dis_ref, b_ref, g_ref, bb_ref, r_ref, o_ref):
    t = jnp.concatenate([p_ref[0], p_ref[1]], axis=-1) * dis_ref[...] + b_ref[...]
    mu = jnp.mean(t, axis=-1, keepdims=True)
    d = t - mu
    var = jnp.mean(d * d, axis=-1, keepdims=True)
    y = d * lax.rsqrt(var + EPS) * g_ref[...] + bb_ref[...]
    o_ref[...] = jnp.maximum(y, 0.0) + r_ref[...]


def _post(p, dis, b, g, bb, r):
    return pl.pallas_call(
        _post_body,
        grid=(N // BR,),
        in_specs=[
            pl.BlockSpec((2, BR, HH), lambda i: (0, i, 0)),
            pl.BlockSpec((BR, 1), lambda i: (i, 0)),
            pl.BlockSpec((1, H), lambda i: (0, 0)),
            pl.BlockSpec((1, H), lambda i: (0, 0)),
            pl.BlockSpec((1, H), lambda i: (0, 0)),
            pl.BlockSpec((BR, H), lambda i: (i, 0)),
        ],
        out_specs=pl.BlockSpec((BR, H), lambda i: (i, 0)),
        out_shape=jax.ShapeDtypeStruct((N, H), jnp.float32),
    )(p, dis, b, g, bb, r)


def _sum_body(h_ref, o_ref):
    @pl.when(pl.program_id(0) == 0)
    def _():
        o_ref[...] = jnp.zeros_like(o_ref)

    o_ref[...] += jnp.sum(h_ref[...], axis=0, keepdims=True)


def _colsum(h):
    return pl.pallas_call(
        _sum_body,
        grid=(N // BR,),
        in_specs=[pl.BlockSpec((BR, H), lambda i: (i, 0))],
        out_specs=pl.BlockSpec((1, H), lambda i: (0, 0)),
        out_shape=jax.ShapeDtypeStruct((1, H), jnp.float32),
    )(h)


def _head_body(m_ref, w1_ref, b1_ref, g_ref, b_ref, w2_ref, b2_ref, o_ref):
    m = m_ref[...] * (1.0 / N)
    t = jnp.dot(m, w1_ref[...], preferred_element_type=jnp.float32) + b1_ref[...]
    mu = jnp.mean(t, axis=-1, keepdims=True)
    d = t - mu
    var = jnp.mean(d * d, axis=-1, keepdims=True)
    y = d * lax.rsqrt(var + EPS) * g_ref[...] + b_ref[...]
    y = jnp.maximum(y, 0.0)
    o_ref[...] = jnp.dot(y, w2_ref[...], preferred_element_type=jnp.float32) + b2_ref[...]


def _head(m, w1, b1, g, b, w2, b2):
    return pl.pallas_call(
        _head_body,
        grid=(1,),
        in_specs=[
            pl.BlockSpec((1, H), lambda i: (0, 0)),
            pl.BlockSpec((H, H), lambda i: (0, 0)),
            pl.BlockSpec((1, H), lambda i: (0, 0)),
            pl.BlockSpec((1, H), lambda i: (0, 0)),
            pl.BlockSpec((1, H), lambda i: (0, 0)),
            pl.BlockSpec((H, OUT), lambda i: (0, 0)),
            pl.BlockSpec((1, OUT), lambda i: (0, 0)),
        ],
        out_specs=pl.BlockSpec((1, OUT), lambda i: (0, 0)),
        out_shape=jax.ShapeDtypeStruct((1, OUT), jnp.float32),
    )(m, w1, b1, g, b, w2, b2)


def kernel(x, edge_index, params):
    p = params
    ei = edge_index.astype(jnp.int32)

    zeros_d = jnp.zeros((ROWS_PER_SUB, DEGW), jnp.float32)
    ones_d = jnp.ones((GROUP, DEGW), jnp.float32)
    ones_n = jnp.ones((N, 1), jnp.float32)

    degp = _deg_sc(ei, ones_d, zeros_d)
    dis = _dis(degp)

    h = _mm(x, p["emb_W"], p["emb_b"][None], ones_n)
    for i in range(NUM_LAYERS):
        if i % 2 == 0 and i > 0:
            r = _mm(h, p["res_W"], p["res_b"][None], ones_n)
        else:
            r = h
        u2 = _mmu(h, p["conv_W"][i], dis)
        agg = _agg_sc(u2, ei)
        h = _post(agg, dis, p["conv_b"][i][None], p["ln_g"][i][None],
                  p["ln_b"][i][None], r)

    hs = _colsum(h)
    out = _head(hs, p["fc1_W"], p["fc1_b"][None], p["fcn_g"][None],
                p["fcn_b"][None], p["fc2_W"], p["fc2_b"][None])
    return out


# speedup vs baseline: 1.5096x; 1.0002x over previous
import functools

import jax
import jax.numpy as jnp
from jax import lax
from jax.experimental import pallas as pl
from jax.experimental.pallas import tpu as pltpu
from jax.experimental.pallas import tpu_sc as plsc

N = 10000
E = 320000
H = 128
OUT = 2
NUM_LAYERS = 4
EPS = 1e-5

GROUP = 128
NGROUPS = E // GROUP
NWORK = 32
ITERS = -(-NGROUPS // NWORK)
NSUB = 16
NPAD = 10240
ROWS_PER_SUB = NPAD // NSUB
DEGW = 16

BR = 400


HH = H // 2

@functools.lru_cache(maxsize=None)
def _make_agg_sc():
    mesh = plsc.VectorSubcoreMesh(core_axis_name="c", subcore_axis_name="s")
    NB = 2
    G2 = 512
    NG2 = E // G2
    GPW = 40

    @functools.partial(
        pl.kernel,
        out_type=jax.ShapeDtypeStruct((2, NPAD, HH), jnp.float32),
        scratch_types=[
            pltpu.VMEM((NB, 2, G2), jnp.int32),
            pltpu.VMEM((NB, G2, HH), jnp.float32),
            pltpu.VMEM_SHARED((NPAD, HH), jnp.float32),
        ] + [pltpu.SemaphoreType.DMA] * (2 * NB),
        mesh=mesh,
        compiler_params=pltpu.CompilerParams(use_tc_tiling_on_sc=False),
    )
    def agg(u_hbm, ei_hbm, out_hbm, idx_v, rows_v, acc_sh, *sems):
        gsems = sems[:NB]
        ssems = sems[NB:]
        c = lax.axis_index("c")
        s = lax.axis_index("s")
        g0 = s * GPW
        nw = jnp.clip(NG2 - g0, 0, GPW)

        stripe = pl.ds(s * ROWS_PER_SUB, ROWS_PER_SUB)
        pltpu.sync_copy(u_hbm.at[c, stripe], acc_sh.at[stripe])
        plsc.subcore_barrier()

        def start(b, i):
            g = g0 + i
            pltpu.sync_copy(ei_hbm.at[:, pl.ds(g * G2, G2)], idx_v.at[b])

            @pl.when(c == 0)
            def _():
                pltpu.async_copy(u_hbm.at[0].at[idx_v.at[b].at[0]],
                                 rows_v.at[b], gsems[b])

            @pl.when(c == 1)
            def _():
                pltpu.async_copy(u_hbm.at[1].at[idx_v.at[b].at[0]],
                                 rows_v.at[b], gsems[b])

        def wait(b):
            pltpu.make_async_copy(
                u_hbm.at[0].at[pl.ds(0, G2)], rows_v.at[b], gsems[b]).wait()

        def scat(b):
            pltpu.async_copy(rows_v.at[b], acc_sh.at[idx_v.at[b].at[1]],
                             ssems[b], add=True)

        def wait_scat(b):
            pltpu.make_async_copy(rows_v.at[b], acc_sh.at[pl.ds(0, G2)],
                                  ssems[b]).wait()

        for j in range(NB):
            @pl.when(j < nw)
            def _(j=j):
                start(j, j)

        def body(k, _):
            base = k * NB
            for b in range(NB):
                i = base + b

                @pl.when(i < nw)
                def _(b=b, i=i):
                    wait(b)
                    scat(b)

                    @pl.when(i + NB < nw)
                    def _(b=b, i=i):
                        wait_scat(b)
                        start(b, i + NB)

            return ()

        lax.fori_loop(0, GPW // NB, body, ())
        for j in range(NB):
            @pl.when(j < jnp.minimum(nw, NB))
            def _(j=j):
                wait_scat(j)
        plsc.subcore_barrier()
        pltpu.sync_copy(acc_sh.at[stripe], out_hbm.at[c, stripe])

    return agg


def _agg_sc(u2, ei):
    return _make_agg_sc()(u2, ei)


@functools.lru_cache(maxsize=None)
def _make_deg_sc():
    mesh = plsc.VectorSubcoreMesh(core_axis_name="c", subcore_axis_name="s")

    @functools.partial(
        pl.kernel,
        out_type=jax.ShapeDtypeStruct((2, NPAD, DEGW), jnp.float32),
        scratch_types=[
            pltpu.VMEM((2, GROUP), jnp.int32),
            pltpu.VMEM((GROUP, DEGW), jnp.float32),
            pltpu.VMEM_SHARED((NPAD, DEGW), jnp.float32),
        ],
        mesh=mesh,
        compiler_params=pltpu.CompilerParams(use_tc_tiling_on_sc=False),
    )
    def degk(ei_hbm, ones_hbm, zeros_hbm, out_hbm, idx_v, ones_v, acc_sh):
        c = lax.axis_index("c")
        s = lax.axis_index("s")
        w = s * 2 + c
        pltpu.sync_copy(zeros_hbm, acc_sh.at[pl.ds(s * ROWS_PER_SUB, ROWS_PER_SUB)])
        pltpu.sync_copy(ones_hbm, ones_v)
        plsc.subcore_barrier()

        def body(i, _):
            g = i * NWORK + w

            @pl.when(g < NGROUPS)
            def _():
                pltpu.sync_copy(ei_hbm.at[:, pl.ds(g * GROUP, GROUP)], idx_v)
                pltpu.sync_copy(ones_v, acc_sh.at[idx_v.at[1]], add=True)

            return ()

        lax.fori_loop(0, ITERS, body, ())
        plsc.subcore_barrier()
        pltpu.sync_copy(
            acc_sh.at[pl.ds(s * ROWS_PER_SUB, ROWS_PER_SUB)],
            out_hbm.at[c, pl.ds(s * ROWS_PER_SUB, ROWS_PER_SUB)],
        )

    return degk


def _deg_sc(ei, ones_d, zeros_d):
    return _make_deg_sc()(ei, ones_d, zeros_d)


def _mm_body(h_ref, w_ref, b_ref, s_ref, o_ref):
    acc = jnp.dot(h_ref[...], w_ref[...], preferred_element_type=jnp.float32)
    o_ref[...] = (acc + b_ref[...]) * s_ref[...]


def _mm(h, w, b, scale):
    return pl.pallas_call(
        _mm_body,
        grid=(N // BR,),
        in_specs=[
            pl.BlockSpec((BR, H), lambda i: (i, 0)),
            pl.BlockSpec((H, H), lambda i: (0, 0)),
            pl.BlockSpec((1, H), lambda i: (0, 0)),
            pl.BlockSpec((BR, 1), lambda i: (i, 0)),
        ],
        out_specs=pl.BlockSpec((BR, H), lambda i: (i, 0)),
        out_shape=jax.ShapeDtypeStruct((N, H), jnp.float32),
    )(h, w, b, scale)


def _mmu_body(h_ref, w_ref, s_ref, o_ref):
    acc = jnp.dot(h_ref[...], w_ref[...], preferred_element_type=jnp.float32)
    acc = acc * s_ref[...]
    o_ref[0] = acc[:, :HH]
    o_ref[1] = acc[:, HH:]


def _mmu(h, w, scale):
    return pl.pallas_call(
        _mmu_body,
        grid=(N // BR,),
        in_specs=[
            pl.BlockSpec((BR, H), lambda i: (i, 0)),
            pl.BlockSpec((H, H), lambda i: (0, 0)),
            pl.BlockSpec((BR, 1), lambda i: (i, 0)),
        ],
        out_specs=pl.BlockSpec((2, BR, HH), lambda i: (0, i, 0)),
        out_shape=jax.ShapeDtypeStruct((2, NPAD, HH), jnp.float32),
    )(h, w, scale)


def _dis_body(d_ref, o_ref):
    deg = 1.0 + d_ref[0, :, 0:1] + d_ref[1, :, 0:1]
    o_ref[...] = lax.rsqrt(deg)


def _dis(degp):
    return pl.pallas_call(
        _dis_body,
        grid=(N // BR,),
        in_specs=[pl.BlockSpec((2, BR, DEGW), lambda i: (0, i, 0))],
        out_specs=pl.BlockSpec((BR, 1), lambda i: (i, 0)),
        out_shape=jax.ShapeDtypeStruct((N, 1), jnp.float32),
    )(degp)


def _post_body(p_ref, dis_ref, b_ref, g_ref, bb_ref, r_ref, o_ref):
    t = jnp.concatenate([p_ref[0], p_ref[1]], axis=-1) * dis_ref[...] + b_ref[...]
    mu = jnp.mean(t, axis=-1, keepdims=True)
    d = t - mu
    var = jnp.mean(d * d, axis=-1, keepdims=True)
    y = d * lax.rsqrt(var + EPS) * g_ref[...] + bb_ref[...]
    o_ref[...] = jnp.maximum(y, 0.0) + r_ref[...]


def _post(p, dis, b, g, bb, r):
    return pl.pallas_call(
        _post_body,
        grid=(N // BR,),
        in_specs=[
            pl.BlockSpec((2, BR, HH), lambda i: (0, i, 0)),
            pl.BlockSpec((BR, 1), lambda i: (i, 0)),
            pl.BlockSpec((1, H), lambda i: (0, 0)),
            pl.BlockSpec((1, H), lambda i: (0, 0)),
            pl.BlockSpec((1, H), lambda i: (0, 0)),
            pl.BlockSpec((BR, H), lambda i: (i, 0)),
        ],
        out_specs=pl.BlockSpec((BR, H), lambda i: (i, 0)),
        out_shape=jax.ShapeDtypeStruct((N, H), jnp.float32),
    )(p, dis, b, g, bb, r)


def _sum_body(h_ref, o_ref):
    @pl.when(pl.program_id(0) == 0)
    def _():
        o_ref[...] = jnp.zeros_like(o_ref)

    o_ref[...] += jnp.sum(h_ref[...], axis=0, keepdims=True)


def _colsum(h):
    return pl.pallas_call(
        _sum_body,
        grid=(N // BR,),
        in_specs=[pl.BlockSpec((BR, H), lambda i: (i, 0))],
        out_specs=pl.BlockSpec((1, H), lambda i: (0, 0)),
        out_shape=jax.ShapeDtypeStruct((1, H), jnp.float32),
    )(h)


def _head_body(m_ref, w1_ref, b1_ref, g_ref, b_ref, w2_ref, b2_ref, o_ref):
    m = m_ref[...] * (1.0 / N)
    t = jnp.dot(m, w1_ref[...], preferred_element_type=jnp.float32) + b1_ref[...]
    mu = jnp.mean(t, axis=-1, keepdims=True)
    d = t - mu
    var = jnp.mean(d * d, axis=-1, keepdims=True)
    y = d * lax.rsqrt(var + EPS) * g_ref[...] + b_ref[...]
    y = jnp.maximum(y, 0.0)
    o_ref[...] = jnp.dot(y, w2_ref[...], preferred_element_type=jnp.float32) + b2_ref[...]


def _head(m, w1, b1, g, b, w2, b2):
    return pl.pallas_call(
        _head_body,
        grid=(1,),
        in_specs=[
            pl.BlockSpec((1, H), lambda i: (0, 0)),
            pl.BlockSpec((H, H), lambda i: (0, 0)),
            pl.BlockSpec((1, H), lambda i: (0, 0)),
            pl.BlockSpec((1, H), lambda i: (0, 0)),
            pl.BlockSpec((1, H), lambda i: (0, 0)),
            pl.BlockSpec((H, OUT), lambda i: (0, 0)),
            pl.BlockSpec((1, OUT), lambda i: (0, 0)),
        ],
        out_specs=pl.BlockSpec((1, OUT), lambda i: (0, 0)),
        out_shape=jax.ShapeDtypeStruct((1, OUT), jnp.float32),
    )(m, w1, b1, g, b, w2, b2)


def kernel(x, edge_index, params):
    p = params
    ei = edge_index.astype(jnp.int32)

    zeros_d = jnp.zeros((ROWS_PER_SUB, DEGW), jnp.float32)
    ones_d = jnp.ones((GROUP, DEGW), jnp.float32)
    ones_n = jnp.ones((N, 1), jnp.float32)

    degp = _deg_sc(ei, ones_d, zeros_d)
    dis = _dis(degp)

    h = _mm(x, p["emb_W"], p["emb_b"][None], ones_n)
    for i in range(NUM_LAYERS):
        if i % 2 == 0 and i > 0:
            r = _mm(h, p["res_W"], p["res_b"][None], ones_n)
        else:
            r = h
        u2 = _mmu(h, p["conv_W"][i], dis)
        agg = _agg_sc(u2, ei)
        h = _post(agg, dis, p["conv_b"][i][None], p["ln_g"][i][None],
                  p["ln_b"][i][None], r)

    hs = _colsum(h)
    out = _head(hs, p["fc1_W"], p["fc1_b"][None], p["fcn_g"][None],
                p["fcn_b"][None], p["fc2_W"], p["fc2_b"][None])
    return out


# speedup vs baseline: 1.6536x; 1.0953x over previous
import functools

import jax
import jax.numpy as jnp
from jax import lax
from jax.experimental import pallas as pl
from jax.experimental.pallas import tpu as pltpu
from jax.experimental.pallas import tpu_sc as plsc

N = 10000
E = 320000
H = 128
OUT = 2
NUM_LAYERS = 4
EPS = 1e-5

GROUP = 128
NGROUPS = E // GROUP
NWORK = 32
ITERS = -(-NGROUPS // NWORK)
NSUB = 16
NPAD = 10240
ROWS_PER_SUB = NPAD // NSUB
DEGW = 16

BR = 400


HH = H // 2

@functools.lru_cache(maxsize=None)
def _make_agg_sc():
    mesh = plsc.VectorSubcoreMesh(core_axis_name="c", subcore_axis_name="s")
    NB = 2
    G2 = 512
    NG2 = E // G2
    GPW = 40

    @functools.partial(
        pl.kernel,
        out_type=jax.ShapeDtypeStruct((2, NPAD, HH), jnp.float32),
        scratch_types=[
            pltpu.VMEM((NB, 2, G2), jnp.int32),
            pltpu.VMEM((NB, G2, HH), jnp.float32),
            pltpu.VMEM_SHARED((NPAD, HH), jnp.float32),
        ] + [pltpu.SemaphoreType.DMA] * (2 * NB),
        mesh=mesh,
        compiler_params=pltpu.CompilerParams(use_tc_tiling_on_sc=False),
    )
    def agg(u_hbm, ei_hbm, out_hbm, idx_v, rows_v, acc_sh, *sems):
        gsems = sems[:NB]
        ssems = sems[NB:]
        c = lax.axis_index("c")
        s = lax.axis_index("s")
        g0 = s * GPW
        nw = jnp.clip(NG2 - g0, 0, GPW)

        stripe = pl.ds(s * ROWS_PER_SUB, ROWS_PER_SUB)
        pltpu.sync_copy(u_hbm.at[c, stripe], acc_sh.at[stripe])
        plsc.subcore_barrier()

        def start(b, i):
            g = g0 + i
            pltpu.sync_copy(ei_hbm.at[:, pl.ds(g * G2, G2)], idx_v.at[b])

            @pl.when(c == 0)
            def _():
                pltpu.async_copy(u_hbm.at[0].at[idx_v.at[b].at[0]],
                                 rows_v.at[b], gsems[b])

            @pl.when(c == 1)
            def _():
                pltpu.async_copy(u_hbm.at[1].at[idx_v.at[b].at[0]],
                                 rows_v.at[b], gsems[b])

        def wait(b):
            pltpu.make_async_copy(
                u_hbm.at[0].at[pl.ds(0, G2)], rows_v.at[b], gsems[b]).wait()

        def scat(b):
            pltpu.async_copy(rows_v.at[b], acc_sh.at[idx_v.at[b].at[1]],
                             ssems[b], add=True)

        def wait_scat(b):
            pltpu.make_async_copy(rows_v.at[b], acc_sh.at[pl.ds(0, G2)],
                                  ssems[b]).wait()

        for j in range(NB):
            @pl.when(j < nw)
            def _(j=j):
                start(j, j)

        def body(k, _):
            base = k * NB
            for b in range(NB):
                i = base + b

                @pl.when(i < nw)
                def _(b=b, i=i):
                    wait(b)
                    scat(b)

                    @pl.when(i + NB < nw)
                    def _(b=b, i=i):
                        wait_scat(b)
                        start(b, i + NB)

            return ()

        lax.fori_loop(0, GPW // NB, body, ())
        for j in range(NB):
            @pl.when(j < jnp.minimum(nw, NB))
            def _(j=j):
                wait_scat(j)
        plsc.subcore_barrier()
        pltpu.sync_copy(acc_sh.at[stripe], out_hbm.at[c, stripe])

    return agg


def _agg_sc(u2, ei):
    return _make_agg_sc()(u2, ei)


@functools.lru_cache(maxsize=None)
def _make_deg_sc():
    mesh = plsc.VectorSubcoreMesh(core_axis_name="c", subcore_axis_name="s")

    @functools.partial(
        pl.kernel,
        out_type=jax.ShapeDtypeStruct((2, NPAD, DEGW), jnp.float32),
        scratch_types=[
            pltpu.VMEM((2, GROUP), jnp.int32),
            pltpu.VMEM((GROUP, DEGW), jnp.float32),
            pltpu.VMEM_SHARED((NPAD, DEGW), jnp.float32),
        ],
        mesh=mesh,
        compiler_params=pltpu.CompilerParams(use_tc_tiling_on_sc=False),
    )
    def degk(ei_hbm, ones_hbm, zeros_hbm, out_hbm, idx_v, ones_v, acc_sh):
        c = lax.axis_index("c")
        s = lax.axis_index("s")
        w = s * 2 + c
        pltpu.sync_copy(zeros_hbm, acc_sh.at[pl.ds(s * ROWS_PER_SUB, ROWS_PER_SUB)])
        pltpu.sync_copy(ones_hbm, ones_v)
        plsc.subcore_barrier()

        def body(i, _):
            g = i * NWORK + w

            @pl.when(g < NGROUPS)
            def _():
                pltpu.sync_copy(ei_hbm.at[:, pl.ds(g * GROUP, GROUP)], idx_v)
                pltpu.sync_copy(ones_v, acc_sh.at[idx_v.at[1]], add=True)

            return ()

        lax.fori_loop(0, ITERS, body, ())
        plsc.subcore_barrier()
        pltpu.sync_copy(
            acc_sh.at[pl.ds(s * ROWS_PER_SUB, ROWS_PER_SUB)],
            out_hbm.at[c, pl.ds(s * ROWS_PER_SUB, ROWS_PER_SUB)],
        )

    return degk


def _deg_sc(ei, ones_d, zeros_d):
    return _make_deg_sc()(ei, ones_d, zeros_d)


def _mm_body(h_ref, w_ref, b_ref, s_ref, o_ref):
    acc = jnp.dot(h_ref[...], w_ref[...], preferred_element_type=jnp.float32)
    o_ref[...] = (acc + b_ref[...]) * s_ref[...]


def _mm(h, w, b, scale):
    return pl.pallas_call(
        _mm_body,
        grid=(N // BR,),
        in_specs=[
            pl.BlockSpec((BR, H), lambda i: (i, 0)),
            pl.BlockSpec((H, H), lambda i: (0, 0)),
            pl.BlockSpec((1, H), lambda i: (0, 0)),
            pl.BlockSpec((BR, 1), lambda i: (i, 0)),
        ],
        out_specs=pl.BlockSpec((BR, H), lambda i: (i, 0)),
        out_shape=jax.ShapeDtypeStruct((N, H), jnp.float32),
    )(h, w, b, scale)


def _mmu_body(h_ref, w_ref, s_ref, o_ref):
    acc = jnp.dot(h_ref[...], w_ref[...], preferred_element_type=jnp.float32)
    acc = acc * s_ref[...]
    o_ref[0] = acc[:, :HH]
    o_ref[1] = acc[:, HH:]


def _mmu(h, w, scale):
    return pl.pallas_call(
        _mmu_body,
        grid=(N // BR,),
        in_specs=[
            pl.BlockSpec((BR, H), lambda i: (i, 0)),
            pl.BlockSpec((H, H), lambda i: (0, 0)),
            pl.BlockSpec((BR, 1), lambda i: (i, 0)),
        ],
        out_specs=pl.BlockSpec((2, BR, HH), lambda i: (0, i, 0)),
        out_shape=jax.ShapeDtypeStruct((2, NPAD, HH), jnp.float32),
    )(h, w, scale)


def _dis_body(d_ref, o_ref):
    deg = 1.0 + d_ref[0, :, 0:1] + d_ref[1, :, 0:1]
    o_ref[...] = lax.rsqrt(deg)


def _dis(degp):
    return pl.pallas_call(
        _dis_body,
        grid=(N // BR,),
        in_specs=[pl.BlockSpec((2, BR, DEGW), lambda i: (0, i, 0))],
        out_specs=pl.BlockSpec((BR, 1), lambda i: (i, 0)),
        out_shape=jax.ShapeDtypeStruct((N, 1), jnp.float32),
    )(degp)


def _post_body(p_ref, dis_ref, b_ref, g_ref, bb_ref, r_ref, o_ref):
    t = jnp.concatenate([p_ref[0], p_ref[1]], axis=-1) * dis_ref[...] + b_ref[...]
    mu = jnp.mean(t, axis=-1, keepdims=True)
    d = t - mu
    var = jnp.mean(d * d, axis=-1, keepdims=True)
    y = d * lax.rsqrt(var + EPS) * g_ref[...] + bb_ref[...]
    o_ref[...] = jnp.maximum(y, 0.0) + r_ref[...]


def _post(p, dis, b, g, bb, r):
    return pl.pallas_call(
        _post_body,
        grid=(N // BR,),
        in_specs=[
            pl.BlockSpec((2, BR, HH), lambda i: (0, i, 0)),
            pl.BlockSpec((BR, 1), lambda i: (i, 0)),
            pl.BlockSpec((1, H), lambda i: (0, 0)),
            pl.BlockSpec((1, H), lambda i: (0, 0)),
            pl.BlockSpec((1, H), lambda i: (0, 0)),
            pl.BlockSpec((BR, H), lambda i: (i, 0)),
        ],
        out_specs=pl.BlockSpec((BR, H), lambda i: (i, 0)),
        out_shape=jax.ShapeDtypeStruct((N, H), jnp.float32),
    )(p, dis, b, g, bb, r)


def _femb_body(x_ref, we_ref, be_ref, d_ref, w0_ref, h_ref, dis_ref, u_ref):
    h = jnp.dot(x_ref[...], we_ref[...], preferred_element_type=jnp.float32) + be_ref[...]
    dis = lax.rsqrt(1.0 + d_ref[0, :, 0:1] + d_ref[1, :, 0:1])
    h_ref[...] = h
    dis_ref[...] = dis
    acc = jnp.dot(h, w0_ref[...], preferred_element_type=jnp.float32) * dis
    u_ref[0] = acc[:, :HH]
    u_ref[1] = acc[:, HH:]


def _femb(x, we, be, degp, w0):
    return pl.pallas_call(
        _femb_body,
        grid=(N // BR,),
        in_specs=[
            pl.BlockSpec((BR, H), lambda i: (i, 0)),
            pl.BlockSpec((H, H), lambda i: (0, 0)),
            pl.BlockSpec((1, H), lambda i: (0, 0)),
            pl.BlockSpec((2, BR, DEGW), lambda i: (0, i, 0)),
            pl.BlockSpec((H, H), lambda i: (0, 0)),
        ],
        out_specs=[
            pl.BlockSpec((BR, H), lambda i: (i, 0)),
            pl.BlockSpec((BR, 1), lambda i: (i, 0)),
            pl.BlockSpec((2, BR, HH), lambda i: (0, i, 0)),
        ],
        out_shape=[
            jax.ShapeDtypeStruct((N, H), jnp.float32),
            jax.ShapeDtypeStruct((N, 1), jnp.float32),
            jax.ShapeDtypeStruct((2, NPAD, HH), jnp.float32),
        ],
    )(x, we, be, degp, w0)


def _ln_relu(p_ref, dis_ref, b_ref, g_ref, bb_ref, r_ref):
    t = jnp.concatenate([p_ref[0], p_ref[1]], axis=-1) * dis_ref[...] + b_ref[...]
    mu = jnp.mean(t, axis=-1, keepdims=True)
    d = t - mu
    var = jnp.mean(d * d, axis=-1, keepdims=True)
    y = d * lax.rsqrt(var + EPS) * g_ref[...] + bb_ref[...]
    return jnp.maximum(y, 0.0) + r_ref[...]


def _fpost_body(p_ref, dis_ref, b_ref, g_ref, bb_ref, r_ref, w_ref,
                h_ref, u_ref):
    h = _ln_relu(p_ref, dis_ref, b_ref, g_ref, bb_ref, r_ref)
    h_ref[...] = h
    acc = jnp.dot(h, w_ref[...], preferred_element_type=jnp.float32) * dis_ref[...]
    u_ref[0] = acc[:, :HH]
    u_ref[1] = acc[:, HH:]


_FPOST_SPECS = dict(
    grid=(N // BR,),
    in_specs=[
        pl.BlockSpec((2, BR, HH), lambda i: (0, i, 0)),
        pl.BlockSpec((BR, 1), lambda i: (i, 0)),
        pl.BlockSpec((1, H), lambda i: (0, 0)),
        pl.BlockSpec((1, H), lambda i: (0, 0)),
        pl.BlockSpec((1, H), lambda i: (0, 0)),
        pl.BlockSpec((BR, H), lambda i: (i, 0)),
        pl.BlockSpec((H, H), lambda i: (0, 0)),
    ],
)


def _fpost(p, dis, b, g, bb, r, w):
    return pl.pallas_call(
        _fpost_body,
        out_specs=[
            pl.BlockSpec((BR, H), lambda i: (i, 0)),
            pl.BlockSpec((2, BR, HH), lambda i: (0, i, 0)),
        ],
        out_shape=[
            jax.ShapeDtypeStruct((N, H), jnp.float32),
            jax.ShapeDtypeStruct((2, NPAD, HH), jnp.float32),
        ],
        **_FPOST_SPECS,
    )(p, dis, b, g, bb, r, w)


def _fpost_res_body(p_ref, dis_ref, b_ref, g_ref, bb_ref, r_ref, w_ref,
                    rw_ref, rb_ref, h_ref, u_ref, r2_ref):
    h = _ln_relu(p_ref, dis_ref, b_ref, g_ref, bb_ref, r_ref)
    h_ref[...] = h
    acc = jnp.dot(h, w_ref[...], preferred_element_type=jnp.float32) * dis_ref[...]
    u_ref[0] = acc[:, :HH]
    u_ref[1] = acc[:, HH:]
    r2_ref[...] = jnp.dot(h, rw_ref[...], preferred_element_type=jnp.float32) + rb_ref[...]


def _fpost_res(p, dis, b, g, bb, r, w, rw, rb):
    sp = dict(_FPOST_SPECS)
    sp["in_specs"] = sp["in_specs"] + [
        pl.BlockSpec((H, H), lambda i: (0, 0)),
        pl.BlockSpec((1, H), lambda i: (0, 0)),
    ]
    return pl.pallas_call(
        _fpost_res_body,
        out_specs=[
            pl.BlockSpec((BR, H), lambda i: (i, 0)),
            pl.BlockSpec((2, BR, HH), lambda i: (0, i, 0)),
            pl.BlockSpec((BR, H), lambda i: (i, 0)),
        ],
        out_shape=[
            jax.ShapeDtypeStruct((N, H), jnp.float32),
            jax.ShapeDtypeStruct((2, NPAD, HH), jnp.float32),
            jax.ShapeDtypeStruct((N, H), jnp.float32),
        ],
        **sp,
    )(p, dis, b, g, bb, r, w, rw, rb)


def _fsum_body(p_ref, dis_ref, b_ref, g_ref, bb_ref, r_ref, o_ref):
    h = _ln_relu(p_ref, dis_ref, b_ref, g_ref, bb_ref, r_ref)

    @pl.when(pl.program_id(0) == 0)
    def _():
        o_ref[...] = jnp.zeros_like(o_ref)

    o_ref[...] += jnp.sum(h, axis=0, keepdims=True)


def _fsum(p, dis, b, g, bb, r):
    sp = dict(_FPOST_SPECS)
    sp["in_specs"] = sp["in_specs"][:-1]
    return pl.pallas_call(
        _fsum_body,
        out_specs=pl.BlockSpec((1, H), lambda i: (0, 0)),
        out_shape=jax.ShapeDtypeStruct((1, H), jnp.float32),
        **sp,
    )(p, dis, b, g, bb, r)


def _sum_body(h_ref, o_ref):
    @pl.when(pl.program_id(0) == 0)
    def _():
        o_ref[...] = jnp.zeros_like(o_ref)

    o_ref[...] += jnp.sum(h_ref[...], axis=0, keepdims=True)


def _colsum(h):
    return pl.pallas_call(
        _sum_body,
        grid=(N // BR,),
        in_specs=[pl.BlockSpec((BR, H), lambda i: (i, 0))],
        out_specs=pl.BlockSpec((1, H), lambda i: (0, 0)),
        out_shape=jax.ShapeDtypeStruct((1, H), jnp.float32),
    )(h)


def _head_body(m_ref, w1_ref, b1_ref, g_ref, b_ref, w2_ref, b2_ref, o_ref):
    m = m_ref[...] * (1.0 / N)
    t = jnp.dot(m, w1_ref[...], preferred_element_type=jnp.float32) + b1_ref[...]
    mu = jnp.mean(t, axis=-1, keepdims=True)
    d = t - mu
    var = jnp.mean(d * d, axis=-1, keepdims=True)
    y = d * lax.rsqrt(var + EPS) * g_ref[...] + b_ref[...]
    y = jnp.maximum(y, 0.0)
    o_ref[...] = jnp.dot(y, w2_ref[...], preferred_element_type=jnp.float32) + b2_ref[...]


def _head(m, w1, b1, g, b, w2, b2):
    return pl.pallas_call(
        _head_body,
        grid=(1,),
        in_specs=[
            pl.BlockSpec((1, H), lambda i: (0, 0)),
            pl.BlockSpec((H, H), lambda i: (0, 0)),
            pl.BlockSpec((1, H), lambda i: (0, 0)),
            pl.BlockSpec((1, H), lambda i: (0, 0)),
            pl.BlockSpec((1, H), lambda i: (0, 0)),
            pl.BlockSpec((H, OUT), lambda i: (0, 0)),
            pl.BlockSpec((1, OUT), lambda i: (0, 0)),
        ],
        out_specs=pl.BlockSpec((1, OUT), lambda i: (0, 0)),
        out_shape=jax.ShapeDtypeStruct((1, OUT), jnp.float32),
    )(m, w1, b1, g, b, w2, b2)


def kernel(x, edge_index, params):
    p = params
    ei = edge_index.astype(jnp.int32)

    ones_d = jnp.ones((GROUP, DEGW), jnp.float32)
    zeros_d = jnp.zeros((ROWS_PER_SUB, DEGW), jnp.float32)

    degp = _deg_sc(ei, ones_d, zeros_d)
    h, dis, u2 = _femb(x, p["emb_W"], p["emb_b"][None], degp, p["conv_W"][0])

    r = h
    for i in range(NUM_LAYERS):
        agg = _agg_sc(u2, ei)
        lnp = (p["conv_b"][i][None], p["ln_g"][i][None], p["ln_b"][i][None])
        if i == NUM_LAYERS - 1:
            hs = _fsum(agg, dis, *lnp, r)
        elif i == 1:
            h, u2, r = _fpost_res(agg, dis, *lnp, r, p["conv_W"][i + 1],
                                  p["res_W"], p["res_b"][None])
        else:
            h, u2 = _fpost(agg, dis, *lnp, r, p["conv_W"][i + 1])
            r = h

    out = _head(hs, p["fc1_W"], p["fc1_b"][None], p["fcn_g"][None],
                p["fcn_b"][None], p["fc2_W"], p["fc2_b"][None])
    return out


# speedup vs baseline: 2.1114x; 1.2769x over previous
import functools

import jax
import jax.numpy as jnp
from jax import lax
from jax.experimental import pallas as pl
from jax.experimental.pallas import tpu as pltpu
from jax.experimental.pallas import tpu_sc as plsc

N = 10000
E = 320000
H = 128
OUT = 2
NUM_LAYERS = 4
EPS = 1e-5

GROUP = 128
NGROUPS = E // GROUP
NWORK = 32
ITERS = -(-NGROUPS // NWORK)
NSUB = 16
NPAD = 10240
ROWS_PER_SUB = NPAD // NSUB
DEGW = 16

BR = 400


HH = H // 2

@functools.lru_cache(maxsize=None)
def _make_agg_sc():
    mesh = plsc.VectorSubcoreMesh(core_axis_name="c", subcore_axis_name="s")
    NB = 2
    G2 = 512
    NG2 = E // G2
    GPW = 40

    @functools.partial(
        pl.kernel,
        out_type=jax.ShapeDtypeStruct((2, NPAD, HH), jnp.bfloat16),
        scratch_types=[
            pltpu.VMEM((NB, 2, G2), jnp.int32),
            pltpu.VMEM((NB, G2, HH), jnp.bfloat16),
            pltpu.VMEM_SHARED((NPAD, HH), jnp.bfloat16),
        ] + [pltpu.SemaphoreType.DMA] * (2 * NB),
        mesh=mesh,
        compiler_params=pltpu.CompilerParams(use_tc_tiling_on_sc=False),
    )
    def agg(u_hbm, ei_hbm, out_hbm, idx_v, rows_v, acc_sh, *sems):
        gsems = sems[:NB]
        ssems = sems[NB:]
        c = lax.axis_index("c")
        s = lax.axis_index("s")
        g0 = s * GPW
        nw = jnp.clip(NG2 - g0, 0, GPW)

        stripe = pl.ds(s * ROWS_PER_SUB, ROWS_PER_SUB)
        pltpu.sync_copy(u_hbm.at[c, stripe], acc_sh.at[stripe])
        plsc.subcore_barrier()

        def start(b, i):
            g = g0 + i
            pltpu.sync_copy(ei_hbm.at[:, pl.ds(g * G2, G2)], idx_v.at[b])

            @pl.when(c == 0)
            def _():
                pltpu.async_copy(u_hbm.at[0].at[idx_v.at[b].at[0]],
                                 rows_v.at[b], gsems[b])

            @pl.when(c == 1)
            def _():
                pltpu.async_copy(u_hbm.at[1].at[idx_v.at[b].at[0]],
                                 rows_v.at[b], gsems[b])

        def wait(b):
            pltpu.make_async_copy(
                u_hbm.at[0].at[pl.ds(0, G2)], rows_v.at[b], gsems[b]).wait()

        def scat(b):
            pltpu.async_copy(rows_v.at[b], acc_sh.at[idx_v.at[b].at[1]],
                             ssems[b], add=True)

        def wait_scat(b):
            pltpu.make_async_copy(rows_v.at[b], acc_sh.at[pl.ds(0, G2)],
                                  ssems[b]).wait()

        for j in range(NB):
            @pl.when(j < nw)
            def _(j=j):
                start(j, j)

        def body(k, _):
            base = k * NB
            for b in range(NB):
                i = base + b

                @pl.when(i < nw)
                def _(b=b, i=i):
                    wait(b)
                    scat(b)

                    @pl.when(i + NB < nw)
                    def _(b=b, i=i):
                        wait_scat(b)
                        start(b, i + NB)

            return ()

        lax.fori_loop(0, GPW // NB, body, ())
        for j in range(NB):
            @pl.when(j < jnp.minimum(nw, NB))
            def _(j=j):
                wait_scat(j)
        plsc.subcore_barrier()
        pltpu.sync_copy(acc_sh.at[stripe], out_hbm.at[c, stripe])

    return agg


def _agg_sc(u2, ei):
    return _make_agg_sc()(u2, ei)


@functools.lru_cache(maxsize=None)
def _make_deg_sc():
    mesh = plsc.VectorSubcoreMesh(core_axis_name="c", subcore_axis_name="s")

    @functools.partial(
        pl.kernel,
        out_type=jax.ShapeDtypeStruct((2, NPAD, DEGW), jnp.float32),
        scratch_types=[
            pltpu.VMEM((2, GROUP), jnp.int32),
            pltpu.VMEM((GROUP, DEGW), jnp.float32),
            pltpu.VMEM_SHARED((NPAD, DEGW), jnp.float32),
        ],
        mesh=mesh,
        compiler_params=pltpu.CompilerParams(use_tc_tiling_on_sc=False),
    )
    def degk(ei_hbm, ones_hbm, zeros_hbm, out_hbm, idx_v, ones_v, acc_sh):
        c = lax.axis_index("c")
        s = lax.axis_index("s")
        w = s * 2 + c
        pltpu.sync_copy(zeros_hbm, acc_sh.at[pl.ds(s * ROWS_PER_SUB, ROWS_PER_SUB)])
        pltpu.sync_copy(ones_hbm, ones_v)
        plsc.subcore_barrier()

        def body(i, _):
            g = i * NWORK + w

            @pl.when(g < NGROUPS)
            def _():
                pltpu.sync_copy(ei_hbm.at[:, pl.ds(g * GROUP, GROUP)], idx_v)
                pltpu.sync_copy(ones_v, acc_sh.at[idx_v.at[1]], add=True)

            return ()

        lax.fori_loop(0, ITERS, body, ())
        plsc.subcore_barrier()
        pltpu.sync_copy(
            acc_sh.at[pl.ds(s * ROWS_PER_SUB, ROWS_PER_SUB)],
            out_hbm.at[c, pl.ds(s * ROWS_PER_SUB, ROWS_PER_SUB)],
        )

    return degk


def _deg_sc(ei, ones_d, zeros_d):
    return _make_deg_sc()(ei, ones_d, zeros_d)


def _mm_body(h_ref, w_ref, b_ref, s_ref, o_ref):
    acc = jnp.dot(h_ref[...], w_ref[...], preferred_element_type=jnp.float32)
    o_ref[...] = (acc + b_ref[...]) * s_ref[...]


def _mm(h, w, b, scale):
    return pl.pallas_call(
        _mm_body,
        grid=(N // BR,),
        in_specs=[
            pl.BlockSpec((BR, H), lambda i: (i, 0)),
            pl.BlockSpec((H, H), lambda i: (0, 0)),
            pl.BlockSpec((1, H), lambda i: (0, 0)),
            pl.BlockSpec((BR, 1), lambda i: (i, 0)),
        ],
        out_specs=pl.BlockSpec((BR, H), lambda i: (i, 0)),
        out_shape=jax.ShapeDtypeStruct((N, H), jnp.float32),
    )(h, w, b, scale)


def _mmu_body(h_ref, w_ref, s_ref, o_ref):
    acc = jnp.dot(h_ref[...], w_ref[...], preferred_element_type=jnp.float32)
    acc = acc * s_ref[...]
    o_ref[0] = acc[:, :HH]
    o_ref[1] = acc[:, HH:]


def _mmu(h, w, scale):
    return pl.pallas_call(
        _mmu_body,
        grid=(N // BR,),
        in_specs=[
            pl.BlockSpec((BR, H), lambda i: (i, 0)),
            pl.BlockSpec((H, H), lambda i: (0, 0)),
            pl.BlockSpec((BR, 1), lambda i: (i, 0)),
        ],
        out_specs=pl.BlockSpec((2, BR, HH), lambda i: (0, i, 0)),
        out_shape=jax.ShapeDtypeStruct((2, NPAD, HH), jnp.float32),
    )(h, w, scale)


def _dis_body(d_ref, o_ref):
    deg = 1.0 + d_ref[0, :, 0:1] + d_ref[1, :, 0:1]
    o_ref[...] = lax.rsqrt(deg)


def _dis(degp):
    return pl.pallas_call(
        _dis_body,
        grid=(N // BR,),
        in_specs=[pl.BlockSpec((2, BR, DEGW), lambda i: (0, i, 0))],
        out_specs=pl.BlockSpec((BR, 1), lambda i: (i, 0)),
        out_shape=jax.ShapeDtypeStruct((N, 1), jnp.float32),
    )(degp)


def _post_body(p_ref, dis_ref, b_ref, g_ref, bb_ref, r_ref, o_ref):
    t = (jnp.concatenate([p_ref[0], p_ref[1]], axis=-1).astype(jnp.float32)
         * dis_ref[...] + b_ref[...])
    mu = jnp.mean(t, axis=-1, keepdims=True)
    d = t - mu
    var = jnp.mean(d * d, axis=-1, keepdims=True)
    y = d * lax.rsqrt(var + EPS) * g_ref[...] + bb_ref[...]
    o_ref[...] = jnp.maximum(y, 0.0) + r_ref[...]


def _post(p, dis, b, g, bb, r):
    return pl.pallas_call(
        _post_body,
        grid=(N // BR,),
        in_specs=[
            pl.BlockSpec((2, BR, HH), lambda i: (0, i, 0)),
            pl.BlockSpec((BR, 1), lambda i: (i, 0)),
            pl.BlockSpec((1, H), lambda i: (0, 0)),
            pl.BlockSpec((1, H), lambda i: (0, 0)),
            pl.BlockSpec((1, H), lambda i: (0, 0)),
            pl.BlockSpec((BR, H), lambda i: (i, 0)),
        ],
        out_specs=pl.BlockSpec((BR, H), lambda i: (i, 0)),
        out_shape=jax.ShapeDtypeStruct((N, H), jnp.float32),
    )(p, dis, b, g, bb, r)


def _femb_body(x_ref, we_ref, be_ref, d_ref, w0_ref, h_ref, dis_ref, u_ref):
    h = jnp.dot(x_ref[...], we_ref[...], preferred_element_type=jnp.float32) + be_ref[...]
    dis = lax.rsqrt(1.0 + d_ref[0, :, 0:1] + d_ref[1, :, 0:1])
    h_ref[...] = h
    dis_ref[...] = dis
    acc = (jnp.dot(h, w0_ref[...], preferred_element_type=jnp.float32) * dis
           ).astype(jnp.bfloat16)
    u_ref[0] = acc[:, :HH]
    u_ref[1] = acc[:, HH:]


def _femb(x, we, be, degp, w0):
    return pl.pallas_call(
        _femb_body,
        grid=(N // BR,),
        in_specs=[
            pl.BlockSpec((BR, H), lambda i: (i, 0)),
            pl.BlockSpec((H, H), lambda i: (0, 0)),
            pl.BlockSpec((1, H), lambda i: (0, 0)),
            pl.BlockSpec((2, BR, DEGW), lambda i: (0, i, 0)),
            pl.BlockSpec((H, H), lambda i: (0, 0)),
        ],
        out_specs=[
            pl.BlockSpec((BR, H), lambda i: (i, 0)),
            pl.BlockSpec((BR, 1), lambda i: (i, 0)),
            pl.BlockSpec((2, BR, HH), lambda i: (0, i, 0)),
        ],
        out_shape=[
            jax.ShapeDtypeStruct((N, H), jnp.float32),
            jax.ShapeDtypeStruct((N, 1), jnp.float32),
            jax.ShapeDtypeStruct((2, NPAD, HH), jnp.bfloat16),
        ],
    )(x, we, be, degp, w0)


def _ln_relu(p_ref, dis_ref, b_ref, g_ref, bb_ref, r_ref):
    t = (jnp.concatenate([p_ref[0], p_ref[1]], axis=-1).astype(jnp.float32)
         * dis_ref[...] + b_ref[...])
    mu = jnp.mean(t, axis=-1, keepdims=True)
    d = t - mu
    var = jnp.mean(d * d, axis=-1, keepdims=True)
    y = d * lax.rsqrt(var + EPS) * g_ref[...] + bb_ref[...]
    return jnp.maximum(y, 0.0) + r_ref[...]


def _fpost_body(p_ref, dis_ref, b_ref, g_ref, bb_ref, r_ref, w_ref,
                h_ref, u_ref):
    h = _ln_relu(p_ref, dis_ref, b_ref, g_ref, bb_ref, r_ref)
    h_ref[...] = h
    acc = (jnp.dot(h, w_ref[...], preferred_element_type=jnp.float32)
           * dis_ref[...]).astype(jnp.bfloat16)
    u_ref[0] = acc[:, :HH]
    u_ref[1] = acc[:, HH:]


_FPOST_SPECS = dict(
    grid=(N // BR,),
    in_specs=[
        pl.BlockSpec((2, BR, HH), lambda i: (0, i, 0)),
        pl.BlockSpec((BR, 1), lambda i: (i, 0)),
        pl.BlockSpec((1, H), lambda i: (0, 0)),
        pl.BlockSpec((1, H), lambda i: (0, 0)),
        pl.BlockSpec((1, H), lambda i: (0, 0)),
        pl.BlockSpec((BR, H), lambda i: (i, 0)),
        pl.BlockSpec((H, H), lambda i: (0, 0)),
    ],
)


def _fpost(p, dis, b, g, bb, r, w):
    return pl.pallas_call(
        _fpost_body,
        out_specs=[
            pl.BlockSpec((BR, H), lambda i: (i, 0)),
            pl.BlockSpec((2, BR, HH), lambda i: (0, i, 0)),
        ],
        out_shape=[
            jax.ShapeDtypeStruct((N, H), jnp.float32),
            jax.ShapeDtypeStruct((2, NPAD, HH), jnp.bfloat16),
        ],
        **_FPOST_SPECS,
    )(p, dis, b, g, bb, r, w)


def _fpost_res_body(p_ref, dis_ref, b_ref, g_ref, bb_ref, r_ref, w_ref,
                    rw_ref, rb_ref, h_ref, u_ref, r2_ref):
    h = _ln_relu(p_ref, dis_ref, b_ref, g_ref, bb_ref, r_ref)
    h_ref[...] = h
    acc = (jnp.dot(h, w_ref[...], preferred_element_type=jnp.float32)
           * dis_ref[...]).astype(jnp.bfloat16)
    u_ref[0] = acc[:, :HH]
    u_ref[1] = acc[:, HH:]
    r2_ref[...] = jnp.dot(h, rw_ref[...], preferred_element_type=jnp.float32) + rb_ref[...]


def _fpost_res(p, dis, b, g, bb, r, w, rw, rb):
    sp = dict(_FPOST_SPECS)
    sp["in_specs"] = sp["in_specs"] + [
        pl.BlockSpec((H, H), lambda i: (0, 0)),
        pl.BlockSpec((1, H), lambda i: (0, 0)),
    ]
    return pl.pallas_call(
        _fpost_res_body,
        out_specs=[
            pl.BlockSpec((BR, H), lambda i: (i, 0)),
            pl.BlockSpec((2, BR, HH), lambda i: (0, i, 0)),
            pl.BlockSpec((BR, H), lambda i: (i, 0)),
        ],
        out_shape=[
            jax.ShapeDtypeStruct((N, H), jnp.float32),
            jax.ShapeDtypeStruct((2, NPAD, HH), jnp.bfloat16),
            jax.ShapeDtypeStruct((N, H), jnp.float32),
        ],
        **sp,
    )(p, dis, b, g, bb, r, w, rw, rb)


def _fsum_body(p_ref, dis_ref, b_ref, g_ref, bb_ref, r_ref, o_ref):
    h = _ln_relu(p_ref, dis_ref, b_ref, g_ref, bb_ref, r_ref)

    @pl.when(pl.program_id(0) == 0)
    def _():
        o_ref[...] = jnp.zeros_like(o_ref)

    o_ref[...] += jnp.sum(h, axis=0, keepdims=True)


def _fsum(p, dis, b, g, bb, r):
    sp = dict(_FPOST_SPECS)
    sp["in_specs"] = sp["in_specs"][:-1]
    return pl.pallas_call(
        _fsum_body,
        out_specs=pl.BlockSpec((1, H), lambda i: (0, 0)),
        out_shape=jax.ShapeDtypeStruct((1, H), jnp.float32),
        **sp,
    )(p, dis, b, g, bb, r)


def _sum_body(h_ref, o_ref):
    @pl.when(pl.program_id(0) == 0)
    def _():
        o_ref[...] = jnp.zeros_like(o_ref)

    o_ref[...] += jnp.sum(h_ref[...], axis=0, keepdims=True)


def _colsum(h):
    return pl.pallas_call(
        _sum_body,
        grid=(N // BR,),
        in_specs=[pl.BlockSpec((BR, H), lambda i: (i, 0))],
        out_specs=pl.BlockSpec((1, H), lambda i: (0, 0)),
        out_shape=jax.ShapeDtypeStruct((1, H), jnp.float32),
    )(h)


def _head_body(m_ref, w1_ref, b1_ref, g_ref, b_ref, w2_ref, b2_ref, o_ref):
    m = m_ref[...] * (1.0 / N)
    t = jnp.dot(m, w1_ref[...], preferred_element_type=jnp.float32) + b1_ref[...]
    mu = jnp.mean(t, axis=-1, keepdims=True)
    d = t - mu
    var = jnp.mean(d * d, axis=-1, keepdims=True)
    y = d * lax.rsqrt(var + EPS) * g_ref[...] + b_ref[...]
    y = jnp.maximum(y, 0.0)
    o_ref[...] = jnp.dot(y, w2_ref[...], preferred_element_type=jnp.float32) + b2_ref[...]


def _head(m, w1, b1, g, b, w2, b2):
    return pl.pallas_call(
        _head_body,
        grid=(1,),
        in_specs=[
            pl.BlockSpec((1, H), lambda i: (0, 0)),
            pl.BlockSpec((H, H), lambda i: (0, 0)),
            pl.BlockSpec((1, H), lambda i: (0, 0)),
            pl.BlockSpec((1, H), lambda i: (0, 0)),
            pl.BlockSpec((1, H), lambda i: (0, 0)),
            pl.BlockSpec((H, OUT), lambda i: (0, 0)),
            pl.BlockSpec((1, OUT), lambda i: (0, 0)),
        ],
        out_specs=pl.BlockSpec((1, OUT), lambda i: (0, 0)),
        out_shape=jax.ShapeDtypeStruct((1, OUT), jnp.float32),
    )(m, w1, b1, g, b, w2, b2)


def kernel(x, edge_index, params):
    p = params
    ei = edge_index.astype(jnp.int32)

    ones_d = jnp.ones((GROUP, DEGW), jnp.float32)
    zeros_d = jnp.zeros((ROWS_PER_SUB, DEGW), jnp.float32)

    degp = _deg_sc(ei, ones_d, zeros_d)
    h, dis, u2 = _femb(x, p["emb_W"], p["emb_b"][None], degp, p["conv_W"][0])

    r = h
    for i in range(NUM_LAYERS):
        agg = _agg_sc(u2, ei)
        lnp = (p["conv_b"][i][None], p["ln_g"][i][None], p["ln_b"][i][None])
        if i == NUM_LAYERS - 1:
            hs = _fsum(agg, dis, *lnp, r)
        elif i == 1:
            h, u2, r = _fpost_res(agg, dis, *lnp, r, p["conv_W"][i + 1],
                                  p["res_W"], p["res_b"][None])
        else:
            h, u2 = _fpost(agg, dis, *lnp, r, p["conv_W"][i + 1])
            r = h

    out = _head(hs, p["fc1_W"], p["fc1_b"][None], p["fcn_g"][None],
                p["fcn_b"][None], p["fc2_W"], p["fc2_b"][None])
    return out
